# Initial kernel scaffold; baseline (speedup 1.0000x reference)
import jax
import jax.numpy as jnp
from jax.experimental import pallas as pl


def kernel(q, W_so, b_so, W_aw, b_aw, W_v, b_v, W_o, b_o, W1, b1, W2, b2, gamma, beta):
    raise NotImplementedError("write your pallas kernel here")


# speedup vs baseline: 1477.2975x; 1477.2975x over previous
import functools

import jax
import jax.numpy as jnp
from jax import lax
from jax.experimental import pallas as pl
from jax.experimental.pallas import tpu as pltpu
from jax.experimental.pallas import tpu_sc as plsc

QD = 128
EMB = 256
NH = 8
HD = 32
NPT = 4
HID = 512
BS = 2
NQ = QD * QD
BSNQ = BS * NQ
S = BSNQ * NH
NCORNER = 16

QB1 = 256
QB3 = 512

NTILES = 32
GPT = (S // 16) // NTILES
CH = 2
NCHUNK = GPT // CH


def _prep_body(q_ref, wv_ref, bv_ref, wcat_ref, bcat_ref,
               val_ref, idx_ref, w_ref):
    b = pl.program_id(0)
    qi = pl.program_id(1)
    qb = q_ref[0]

    val_ref[0] = jnp.dot(qb, wv_ref[...], preferred_element_type=jnp.float32) + bv_ref[0]

    cat = jnp.dot(qb, wcat_ref[...], preferred_element_type=jnp.float32) + bcat_ref[0]
    oxa = cat[:, 0:32]
    oya = cat[:, 32:64]
    awa = cat[:, 64:96]

    a = [awa[:, 8 * p:8 * p + 8] for p in range(NPT)]
    m = jnp.maximum(jnp.maximum(a[0], a[1]), jnp.maximum(a[2], a[3]))
    e = [jnp.exp(x - m) for x in a]
    rs = 1.0 / (e[0] + e[1] + e[2] + e[3])

    qrow = qi * QB1 + lax.broadcasted_iota(jnp.int32, (QB1, 1), 0)
    irow = qrow // QD
    jcol = qrow - irow * QD
    scale = QD / (QD - 1.0)
    bx = jcol.astype(jnp.float32) * scale - 0.5
    by = irow.astype(jnp.float32) * scale - 0.5
    bofs = b * NQ

    idx_parts = []
    w_parts = []
    for p in range(NPT):
        x = bx + oxa[:, 8 * p:8 * p + 8]
        y = by + oya[:, 8 * p:8 * p + 8]
        awp = e[p] * rs
        x0f = jnp.floor(x)
        y0f = jnp.floor(y)
        fx = x - x0f
        fy = y - y0f
        vx0 = ((x0f >= 0.0) & (x0f <= QD - 1.0)).astype(jnp.float32)
        vx1 = ((x0f >= -1.0) & (x0f <= QD - 2.0)).astype(jnp.float32)
        vy0 = ((y0f >= 0.0) & (y0f <= QD - 1.0)).astype(jnp.float32)
        vy1 = ((y0f >= -1.0) & (y0f <= QD - 2.0)).astype(jnp.float32)
        cx0 = jnp.clip(x0f, 0.0, QD - 1.0).astype(jnp.int32)
        cx1 = jnp.clip(x0f + 1.0, 0.0, QD - 1.0).astype(jnp.int32)
        cy0 = jnp.clip(y0f, 0.0, QD - 1.0).astype(jnp.int32)
        cy1 = jnp.clip(y0f + 1.0, 0.0, QD - 1.0).astype(jnp.int32)
        hcol = lax.broadcasted_iota(jnp.int32, (QB1, 8), 1)
        r00 = (bofs + cy0 * QD + cx0) * NH + hcol
        r01 = (bofs + cy1 * QD + cx0) * NH + hcol
        r10 = (bofs + cy0 * QD + cx1) * NH + hcol
        r11 = (bofs + cy1 * QD + cx1) * NH + hcol
        gx0 = 1.0 - fx
        gy0 = 1.0 - fy
        w00 = gx0 * gy0 * vx0 * vy0 * awp
        w01 = gx0 * fy * vx0 * vy1 * awp
        w10 = fx * gy0 * vx1 * vy0 * awp
        w11 = fx * fy * vx1 * vy1 * awp
        idx_parts += [r00, r01, r10, r11]
        w_parts += [w00, w01, w10, w11]

    idx_ref[...] = jnp.concatenate(idx_parts, axis=1)
    w_ref[...] = jnp.concatenate(w_parts, axis=1)


def _prep_call(q, W_v, b_v, W_cat, b_cat):
    nqb = NQ // QB1
    return pl.pallas_call(
        _prep_body,
        grid=(BS, nqb),
        in_specs=[
            pl.BlockSpec((1, QB1, EMB), lambda b, qi: (b, qi, 0)),
            pl.BlockSpec((EMB, EMB), lambda b, qi: (0, 0)),
            pl.BlockSpec((1, EMB), lambda b, qi: (0, 0)),
            pl.BlockSpec((EMB, 96), lambda b, qi: (0, 0)),
            pl.BlockSpec((1, 96), lambda b, qi: (0, 0)),
        ],
        out_specs=[
            pl.BlockSpec((1, QB1, EMB), lambda b, qi: (b, qi, 0)),
            pl.BlockSpec((QB1, 128), lambda b, qi: (b * (NQ // QB1) + qi, 0)),
            pl.BlockSpec((QB1, 128), lambda b, qi: (b * (NQ // QB1) + qi, 0)),
        ],
        out_shape=[
            jax.ShapeDtypeStruct((BS, NQ, EMB), jnp.float32),
            jax.ShapeDtypeStruct((BSNQ, 128), jnp.int32),
            jax.ShapeDtypeStruct((BSNQ, 128), jnp.float32),
        ],
    )(q, W_v, b_v, W_cat, b_cat)


def _sample_body(idxr, wr, table, out, idx_v, w_v, rows_v, out_v, semg):
    wid = lax.axis_index("s") * 2 + lax.axis_index("c")
    l16 = lax.broadcasted_iota(jnp.int32, (16,), 0)
    lhi = lax.shift_right_logical(l16, 3)
    llo = lax.bitwise_and(l16, 7)

    def chunk(ci, carry):
        gb = wid * GPT + ci * CH
        pltpu.sync_copy(idxr.at[pl.ds(gb * 2, CH * 2)], idx_v)
        pltpu.sync_copy(wr.at[pl.ds(gb * 2, CH * 2)], w_v)
        cps = [pltpu.async_copy(table.at[idx_v.at[j]],
                                rows_v.at[pl.ds(j * 128, 128)], semg)
               for j in range(CH * 2)]
        for cp in cps:
            cp.wait()
        for gl in range(CH):
            roww = gl * 2 + lhi
            wk = [plsc.load_gather(w_v, [roww, k * 8 + llo]) for k in range(NCORNER)]
            rowd = [gl * 256 + lhi * 128 + k * 8 + llo for k in range(NCORNER)]
            outrow = gl * 16 + l16

            def cbody(c, carry):
                cvec = jnp.full((16,), c, jnp.int32)
                acc = wk[0] * plsc.load_gather(rows_v, [rowd[0], cvec])
                for k in range(1, NCORNER):
                    acc = acc + wk[k] * plsc.load_gather(rows_v, [rowd[k], cvec])
                plsc.store_scatter(out_v, [outrow, cvec], acc)
                return carry

            lax.fori_loop(0, HD, cbody, 0)
        pltpu.sync_copy(out_v, out.at[pl.ds(gb * 16, CH * 16)])
        return carry

    lax.fori_loop(0, NCHUNK, chunk, 0)


@functools.lru_cache(maxsize=1)
def _get_sample_sc():
    return pl.kernel(
        _sample_body,
        out_type=jax.ShapeDtypeStruct((S, HD), jnp.float32),
        mesh=plsc.VectorSubcoreMesh(core_axis_name="c", subcore_axis_name="s"),
        compiler_params=pltpu.CompilerParams(needs_layout_passes=False,
                                             use_tc_tiling_on_sc=False),
        scratch_types=[
            pltpu.VMEM((CH * 2, 128), jnp.int32),
            pltpu.VMEM((CH * 2, 128), jnp.float32),
            pltpu.VMEM((CH * 256, HD), jnp.float32),
            pltpu.VMEM((CH * 16, HD), jnp.float32),
            pltpu.SemaphoreType.DMA,
        ],
    )


def _sample_sc(idxf, wf, table):
    return _get_sample_sc()(idxf, wf, table)


def _post_body(s_ref, q_ref, wo_ref, bo_ref, w1_ref, b1_ref, w2_ref, b2_ref,
               g_ref, be_ref, o_ref):
    sb = s_ref[...]
    qb = q_ref[...]
    x = (jnp.dot(sb, wo_ref[...], preferred_element_type=jnp.float32)
         + bo_ref[0] + 2.0 * qb)
    h1 = jnp.maximum(
        jnp.dot(x, w1_ref[...], preferred_element_type=jnp.float32) + b1_ref[0], 0.0)
    t = jnp.dot(h1, w2_ref[...], preferred_element_type=jnp.float32) + b2_ref[0]
    mu = jnp.mean(t, axis=-1, keepdims=True)
    d = t - mu
    var = jnp.mean(d * d, axis=-1, keepdims=True)
    hn = d * lax.rsqrt(var + 1e-5) * g_ref[0] + be_ref[0]
    o_ref[...] = x + hn


def _post_call(sampled, qf, W_o, b_o, W1, b1, W2, b2, gamma, beta):
    nblk = BSNQ // QB3
    full = lambda i: (0, 0)
    return pl.pallas_call(
        _post_body,
        grid=(nblk,),
        in_specs=[
            pl.BlockSpec((QB3, EMB), lambda i: (i, 0)),
            pl.BlockSpec((QB3, EMB), lambda i: (i, 0)),
            pl.BlockSpec((EMB, EMB), full),
            pl.BlockSpec((1, EMB), full),
            pl.BlockSpec((EMB, HID), full),
            pl.BlockSpec((1, HID), full),
            pl.BlockSpec((HID, EMB), full),
            pl.BlockSpec((1, EMB), full),
            pl.BlockSpec((1, EMB), full),
            pl.BlockSpec((1, EMB), full),
        ],
        out_specs=pl.BlockSpec((QB3, EMB), lambda i: (i, 0)),
        out_shape=jax.ShapeDtypeStruct((BSNQ, EMB), jnp.float32),
    )(sampled, qf, W_o, b_o, W1, b1, W2, b2, gamma, beta)


def kernel(q, W_so, b_so, W_aw, b_aw, W_v, b_v, W_o, b_o, W1, b1, W2, b2, gamma, beta):
    wso = W_so.reshape(EMB, NH, NPT, 2)
    bso = b_so.reshape(NH, NPT, 2)
    w_sox = wso[..., 0].transpose(0, 2, 1).reshape(EMB, NH * NPT)
    w_soy = wso[..., 1].transpose(0, 2, 1).reshape(EMB, NH * NPT)
    b_sox = bso[..., 0].transpose(1, 0).reshape(NH * NPT)
    b_soy = bso[..., 1].transpose(1, 0).reshape(NH * NPT)
    w_awr = W_aw.reshape(EMB, NH, NPT).transpose(0, 2, 1).reshape(EMB, NH * NPT)
    b_awr = b_aw.reshape(NH, NPT).transpose(1, 0).reshape(NH * NPT)
    w_cat = jnp.concatenate([w_sox, w_soy, w_awr], axis=1)
    b_cat = jnp.concatenate([b_sox, b_soy, b_awr], axis=0).reshape(1, 96)

    value, idxf, wf = _prep_call(q, W_v, b_v.reshape(1, EMB), w_cat, b_cat)
    table = value.reshape(S, HD)
    sampled = _sample_sc(idxf, wf, table)
    out = _post_call(sampled.reshape(BSNQ, EMB), q.reshape(BSNQ, EMB),
                     W_o, b_o.reshape(1, EMB), W1, b1.reshape(1, HID),
                     W2, b2.reshape(1, EMB), gamma.reshape(1, EMB),
                     beta.reshape(1, EMB))
    return out.reshape(BS, NQ, EMB)


# speedup vs baseline: 1687.0462x; 1.1420x over previous
import functools

import jax
import jax.numpy as jnp
from jax import lax
from jax.experimental import pallas as pl
from jax.experimental.pallas import tpu as pltpu
from jax.experimental.pallas import tpu_sc as plsc

QD = 128
EMB = 256
NH = 8
HD = 32
NPT = 4
HID = 512
BS = 2
NQ = QD * QD
BSNQ = BS * NQ
S = BSNQ * NH
NCORNER = 16

QB1 = 256
QB3 = 512

NTILES = 32
GPT = (S // 16) // NTILES
CH = 4
NCHUNK = GPT // CH


def _prep_body(q_ref, wv_ref, bv_ref, wcat_ref, bcat_ref,
               val_ref, idx_ref, w_ref):
    b = pl.program_id(0)
    qi = pl.program_id(1)
    qb = q_ref[0]

    val_ref[0] = jnp.dot(qb, wv_ref[...], preferred_element_type=jnp.float32) + bv_ref[0]

    cat = jnp.dot(qb, wcat_ref[...], preferred_element_type=jnp.float32) + bcat_ref[0]
    oxa = cat[:, 0:32]
    oya = cat[:, 32:64]
    awa = cat[:, 64:96]

    a = [awa[:, 8 * p:8 * p + 8] for p in range(NPT)]
    m = jnp.maximum(jnp.maximum(a[0], a[1]), jnp.maximum(a[2], a[3]))
    e = [jnp.exp(x - m) for x in a]
    rs = 1.0 / (e[0] + e[1] + e[2] + e[3])

    qrow = qi * QB1 + lax.broadcasted_iota(jnp.int32, (QB1, 1), 0)
    irow = qrow // QD
    jcol = qrow - irow * QD
    scale = QD / (QD - 1.0)
    bx = jcol.astype(jnp.float32) * scale - 0.5
    by = irow.astype(jnp.float32) * scale - 0.5
    bofs = b * NQ

    idx_parts = []
    w_parts = []
    for p in range(NPT):
        x = bx + oxa[:, 8 * p:8 * p + 8]
        y = by + oya[:, 8 * p:8 * p + 8]
        awp = e[p] * rs
        x0f = jnp.floor(x)
        y0f = jnp.floor(y)
        fx = x - x0f
        fy = y - y0f
        vx0 = ((x0f >= 0.0) & (x0f <= QD - 1.0)).astype(jnp.float32)
        vx1 = ((x0f >= -1.0) & (x0f <= QD - 2.0)).astype(jnp.float32)
        vy0 = ((y0f >= 0.0) & (y0f <= QD - 1.0)).astype(jnp.float32)
        vy1 = ((y0f >= -1.0) & (y0f <= QD - 2.0)).astype(jnp.float32)
        cx0 = jnp.clip(x0f, 0.0, QD - 1.0).astype(jnp.int32)
        cx1 = jnp.clip(x0f + 1.0, 0.0, QD - 1.0).astype(jnp.int32)
        cy0 = jnp.clip(y0f, 0.0, QD - 1.0).astype(jnp.int32)
        cy1 = jnp.clip(y0f + 1.0, 0.0, QD - 1.0).astype(jnp.int32)
        hcol = lax.broadcasted_iota(jnp.int32, (QB1, 8), 1)
        r00 = (bofs + cy0 * QD + cx0) * NH + hcol
        r01 = (bofs + cy1 * QD + cx0) * NH + hcol
        r10 = (bofs + cy0 * QD + cx1) * NH + hcol
        r11 = (bofs + cy1 * QD + cx1) * NH + hcol
        gx0 = 1.0 - fx
        gy0 = 1.0 - fy
        w00 = gx0 * gy0 * vx0 * vy0 * awp
        w01 = gx0 * fy * vx0 * vy1 * awp
        w10 = fx * gy0 * vx1 * vy0 * awp
        w11 = fx * fy * vx1 * vy1 * awp
        idx_parts += [r00, r01, r10, r11]
        w_parts += [w00, w01, w10, w11]

    idx_ref[...] = jnp.concatenate(idx_parts, axis=1)
    w_ref[...] = jnp.concatenate(w_parts, axis=1)


def _prep_call(q, W_v, b_v, W_cat, b_cat):
    nqb = NQ // QB1
    return pl.pallas_call(
        _prep_body,
        grid=(BS, nqb),
        in_specs=[
            pl.BlockSpec((1, QB1, EMB), lambda b, qi: (b, qi, 0)),
            pl.BlockSpec((EMB, EMB), lambda b, qi: (0, 0)),
            pl.BlockSpec((1, EMB), lambda b, qi: (0, 0)),
            pl.BlockSpec((EMB, 96), lambda b, qi: (0, 0)),
            pl.BlockSpec((1, 96), lambda b, qi: (0, 0)),
        ],
        out_specs=[
            pl.BlockSpec((1, QB1, EMB), lambda b, qi: (b, qi, 0)),
            pl.BlockSpec((QB1, 128), lambda b, qi: (b * (NQ // QB1) + qi, 0)),
            pl.BlockSpec((QB1, 128), lambda b, qi: (b * (NQ // QB1) + qi, 0)),
        ],
        out_shape=[
            jax.ShapeDtypeStruct((BS, NQ, EMB), jnp.float32),
            jax.ShapeDtypeStruct((BSNQ, 128), jnp.int32),
            jax.ShapeDtypeStruct((BSNQ, 128), jnp.float32),
        ],
    )(q, W_v, b_v, W_cat, b_cat)


def _tree_sum(xs):
    while len(xs) > 1:
        xs = [xs[i] + xs[i + 1] for i in range(0, len(xs) - 1, 2)] + (
            [xs[-1]] if len(xs) % 2 else [])
    return xs[0]


def _sample_body(idxr, wr, table, out, idx_v, w_v, rows_v, out_v, semg):
    wid = lax.axis_index("s") * 2 + lax.axis_index("c")
    l16 = lax.broadcasted_iota(jnp.int32, (16,), 0)
    lhi = lax.shift_right_logical(l16, 3)
    llo = lax.bitwise_and(l16, 7)
    g0 = wid * GPT

    def fetch(ci, s):
        gb = g0 + ci * CH
        pltpu.sync_copy(idxr.at[pl.ds(gb * 2, CH * 2)], idx_v.at[s])
        pltpu.sync_copy(wr.at[pl.ds(gb * 2, CH * 2)], w_v.at[s])
        for j in range(CH * 2):
            pltpu.async_copy(table.at[idx_v.at[s, j]],
                             rows_v.at[s, pl.ds(j * 128, 128)], semg.at[s])

    def drain(s):
        pltpu.make_async_copy(table.at[pl.ds(0, CH * 256)], rows_v.at[s],
                              semg.at[s]).wait()

    def compute(ci, s):
        gb = g0 + ci * CH
        for gl in range(CH):
            roww = gl * 2 + lhi
            wk = [plsc.load_gather(w_v.at[s], [roww, k * 8 + llo])
                  for k in range(NCORNER)]
            rowd = [gl * 256 + lhi * 128 + k * 8 + llo for k in range(NCORNER)]
            outrow = gl * 16 + l16

            def cbody(ch2, carry):
                for dc in range(2):
                    cvec = ch2 * 2 + dc + l16 * 0
                    prods = [wk[k] * plsc.load_gather(rows_v.at[s],
                                                      [rowd[k], cvec])
                             for k in range(NCORNER)]
                    plsc.store_scatter(out_v.at[s], [outrow, cvec],
                                       _tree_sum(prods))
                return carry

            lax.fori_loop(0, HD // 2, cbody, 0)
        pltpu.sync_copy(out_v.at[s], out.at[pl.ds(gb * 16, CH * 16)])

    fetch(0, 0)

    def chunk2(i, carry):
        ci = i * 2
        fetch(ci + 1, 1)
        drain(0)
        compute(ci, 0)

        @pl.when(ci + 2 < NCHUNK)
        def _():
            fetch(ci + 2, 0)

        drain(1)
        compute(ci + 1, 1)
        return carry

    lax.fori_loop(0, NCHUNK // 2, chunk2, 0)


@functools.lru_cache(maxsize=1)
def _get_sample_sc():
    return pl.kernel(
        _sample_body,
        out_type=jax.ShapeDtypeStruct((S, HD), jnp.float32),
        mesh=plsc.VectorSubcoreMesh(core_axis_name="c", subcore_axis_name="s"),
        compiler_params=pltpu.CompilerParams(needs_layout_passes=False,
                                             use_tc_tiling_on_sc=False),
        scratch_types=[
            pltpu.VMEM((2, CH * 2, 128), jnp.int32),
            pltpu.VMEM((2, CH * 2, 128), jnp.float32),
            pltpu.VMEM((2, CH * 256, HD), jnp.float32),
            pltpu.VMEM((2, CH * 16, HD), jnp.float32),
            pltpu.SemaphoreType.DMA((2,)),
        ],
    )


def _sample_sc(idxf, wf, table):
    return _get_sample_sc()(idxf, wf, table)


def _post_body(s_ref, q_ref, wo_ref, bo_ref, w1_ref, b1_ref, w2_ref, b2_ref,
               g_ref, be_ref, o_ref):
    sb = s_ref[...]
    qb = q_ref[...]
    x = (jnp.dot(sb, wo_ref[...], preferred_element_type=jnp.float32)
         + bo_ref[0] + 2.0 * qb)
    h1 = jnp.maximum(
        jnp.dot(x, w1_ref[...], preferred_element_type=jnp.float32) + b1_ref[0], 0.0)
    t = jnp.dot(h1, w2_ref[...], preferred_element_type=jnp.float32) + b2_ref[0]
    mu = jnp.mean(t, axis=-1, keepdims=True)
    d = t - mu
    var = jnp.mean(d * d, axis=-1, keepdims=True)
    hn = d * lax.rsqrt(var + 1e-5) * g_ref[0] + be_ref[0]
    o_ref[...] = x + hn


def _post_call(sampled, qf, W_o, b_o, W1, b1, W2, b2, gamma, beta):
    nblk = BSNQ // QB3
    full = lambda i: (0, 0)
    return pl.pallas_call(
        _post_body,
        grid=(nblk,),
        in_specs=[
            pl.BlockSpec((QB3, EMB), lambda i: (i, 0)),
            pl.BlockSpec((QB3, EMB), lambda i: (i, 0)),
            pl.BlockSpec((EMB, EMB), full),
            pl.BlockSpec((1, EMB), full),
            pl.BlockSpec((EMB, HID), full),
            pl.BlockSpec((1, HID), full),
            pl.BlockSpec((HID, EMB), full),
            pl.BlockSpec((1, EMB), full),
            pl.BlockSpec((1, EMB), full),
            pl.BlockSpec((1, EMB), full),
        ],
        out_specs=pl.BlockSpec((QB3, EMB), lambda i: (i, 0)),
        out_shape=jax.ShapeDtypeStruct((BSNQ, EMB), jnp.float32),
    )(sampled, qf, W_o, b_o, W1, b1, W2, b2, gamma, beta)


def kernel(q, W_so, b_so, W_aw, b_aw, W_v, b_v, W_o, b_o, W1, b1, W2, b2, gamma, beta):
    wso = W_so.reshape(EMB, NH, NPT, 2)
    bso = b_so.reshape(NH, NPT, 2)
    w_sox = wso[..., 0].transpose(0, 2, 1).reshape(EMB, NH * NPT)
    w_soy = wso[..., 1].transpose(0, 2, 1).reshape(EMB, NH * NPT)
    b_sox = bso[..., 0].transpose(1, 0).reshape(NH * NPT)
    b_soy = bso[..., 1].transpose(1, 0).reshape(NH * NPT)
    w_awr = W_aw.reshape(EMB, NH, NPT).transpose(0, 2, 1).reshape(EMB, NH * NPT)
    b_awr = b_aw.reshape(NH, NPT).transpose(1, 0).reshape(NH * NPT)
    w_cat = jnp.concatenate([w_sox, w_soy, w_awr], axis=1)
    b_cat = jnp.concatenate([b_sox, b_soy, b_awr], axis=0).reshape(1, 96)

    value, idxf, wf = _prep_call(q, W_v, b_v.reshape(1, EMB), w_cat, b_cat)
    table = value.reshape(S, HD)
    sampled = _sample_sc(idxf, wf, table)
    out = _post_call(sampled.reshape(BSNQ, EMB), q.reshape(BSNQ, EMB),
                     W_o, b_o.reshape(1, EMB), W1, b1.reshape(1, HID),
                     W2, b2.reshape(1, EMB), gamma.reshape(1, EMB),
                     beta.reshape(1, EMB))
    return out.reshape(BS, NQ, EMB)
